# baseline (speedup 1.0000x reference)
import functools

import jax
import jax.numpy as jnp
from jax import lax
from jax.experimental import pallas as pl
from jax.experimental.pallas import tpu as pltpu
from jax.experimental.pallas import tpu_sc as plsc

BATCH = 4
SEQ = 8192
DIM = 768

NC = 2
NS = 16
NW = NC * NS
POS_PER_W = SEQ // NW
R = 32
NSUB = POS_PER_W // R
LANES = 16


def _add_chunk(buf, pos):
    @plsc.parallel_loop(0, R, step=1)
    def body(r):
        for c in range(0, DIM, LANES):
            plsc.addupdate(buf.at[r, pl.ds(c, LANES)],
                           pos[r, pl.ds(c, LANES)])


def _sc_kernel(x_hbm, w_hbm, out_hbm,
               buf0, buf1, buf2, buf3, pos_v,
               sem_ld0, sem_ld1, sem_ld2, sem_ld3,
               sem_st0, sem_st1, sem_st2, sem_st3, sem_p):
    w = lax.axis_index("s") * NC + lax.axis_index("c")
    base = w * POS_PER_W

    bufs = (buf0, buf1, buf2, buf3)
    ld_sems = (sem_ld0, sem_ld1, sem_ld2, sem_ld3)
    st_sems = (sem_st0, sem_st1, sem_st2, sem_st3)

    def wait_ld(k):
        pltpu.make_async_copy(x_hbm.at[0, pl.ds(0, R)], bufs[k],
                              ld_sems[k]).wait()

    def wait_st(k):
        pltpu.make_async_copy(bufs[k], out_hbm.at[0, pl.ds(0, R)],
                              st_sems[k]).wait()

    def wait_pos():
        pltpu.make_async_copy(w_hbm.at[pl.ds(0, R)], pos_v, sem_p).wait()

    pltpu.async_copy(w_hbm.at[pl.ds(base, R)], pos_v, sem_p)
    pltpu.async_copy(x_hbm.at[0, pl.ds(base, R)], bufs[0], ld_sems[0])
    pltpu.async_copy(x_hbm.at[1, pl.ds(base, R)], bufs[1], ld_sems[1])

    def t_body(t, carry):
        row0 = base + t * R
        wait_pos()
        for b in range(BATCH):
            wait_ld(b)
            _add_chunk(bufs[b], pos_v)
            if b == BATCH - 1:
                @pl.when(t + 1 < NSUB)
                def _pos_next():
                    pltpu.async_copy(w_hbm.at[pl.ds(row0 + R, R)], pos_v,
                                     sem_p)
            pltpu.async_copy(bufs[b], out_hbm.at[b, pl.ds(row0, R)],
                             st_sems[b])
            if b < 2:
                j = b + 2

                @pl.when(t > 0)
                def _free():
                    wait_st(j)
                pltpu.async_copy(x_hbm.at[j, pl.ds(row0, R)], bufs[j],
                                 ld_sems[j])
            else:
                j = b - 2

                @pl.when(t + 1 < NSUB)
                def _next():
                    wait_st(j)
                    pltpu.async_copy(x_hbm.at[j, pl.ds(row0 + R, R)],
                                     bufs[j], ld_sems[j])
        return carry

    lax.fori_loop(0, NSUB, t_body, 0)
    for k in range(BATCH):
        wait_st(k)


@jax.jit
def kernel(x, pos_embed_weight):
    mesh = plsc.VectorSubcoreMesh(core_axis_name="c", subcore_axis_name="s",
                                  num_cores=NC, num_subcores=NS)
    run = functools.partial(
        pl.kernel,
        out_type=jax.ShapeDtypeStruct((BATCH, SEQ, DIM), jnp.float32),
        mesh=mesh,
        scratch_types=[
            pltpu.VMEM((R, DIM), jnp.float32),
            pltpu.VMEM((R, DIM), jnp.float32),
            pltpu.VMEM((R, DIM), jnp.float32),
            pltpu.VMEM((R, DIM), jnp.float32),
            pltpu.VMEM((R, DIM), jnp.float32),
            pltpu.SemaphoreType.DMA,
            pltpu.SemaphoreType.DMA,
            pltpu.SemaphoreType.DMA,
            pltpu.SemaphoreType.DMA,
            pltpu.SemaphoreType.DMA,
            pltpu.SemaphoreType.DMA,
            pltpu.SemaphoreType.DMA,
            pltpu.SemaphoreType.DMA,
            pltpu.SemaphoreType.DMA,
        ],
    )(_sc_kernel)
    return run(x, pos_embed_weight)


# speedup vs baseline: 1.0732x; 1.0732x over previous
import functools

import jax
import jax.numpy as jnp
from jax import lax
from jax.experimental import pallas as pl
from jax.experimental.pallas import tpu as pltpu
from jax.experimental.pallas import tpu_sc as plsc

BATCH = 4
SEQ = 8192
DIM = 768

NC = 2
NS = 16
NW = NC * NS
POS_PER_W = SEQ // NW
R = 32
NSUB = POS_PER_W // R
LANES = 16


def _add_chunk(buf, pos):
    @plsc.parallel_loop(0, R, step=1)
    def body(r):
        for c in range(0, DIM, LANES):
            plsc.addupdate(buf.at[r, pl.ds(c, LANES)],
                           pos[r, pl.ds(c, LANES)])


def _sc_kernel(x_hbm, w_hbm, out_hbm,
               buf0, buf1, buf2, buf3, pos_v,
               sem_ld0, sem_ld1, sem_ld2, sem_ld3,
               sem_st0, sem_st1, sem_st2, sem_st3, sem_p):
    w = lax.axis_index("s") * NC + lax.axis_index("c")
    base = w * POS_PER_W

    bufs = (buf0, buf1, buf2, buf3)
    ld_sems = (sem_ld0, sem_ld1, sem_ld2, sem_ld3)
    st_sems = (sem_st0, sem_st1, sem_st2, sem_st3)

    def wait_ld(k):
        pltpu.make_async_copy(x_hbm.at[0, pl.ds(0, R)], bufs[k],
                              ld_sems[k]).wait()

    def wait_st(k):
        pltpu.make_async_copy(bufs[k], out_hbm.at[0, pl.ds(0, R)],
                              st_sems[k]).wait()

    def wait_pos():
        pltpu.make_async_copy(w_hbm.at[pl.ds(0, R)], pos_v, sem_p).wait()

    pltpu.async_copy(w_hbm.at[pl.ds(base, R)], pos_v, sem_p)
    pltpu.async_copy(x_hbm.at[0, pl.ds(base, R)], bufs[0], ld_sems[0])
    pltpu.async_copy(x_hbm.at[1, pl.ds(base, R)], bufs[1], ld_sems[1])

    def t_body(t, carry):
        row0 = base + t * R
        wait_pos()
        for b in range(BATCH):
            wait_ld(b)
            if b < 2:
                j = b + 2

                @pl.when(t > 0)
                def _free():
                    wait_st(j)
                pltpu.async_copy(x_hbm.at[j, pl.ds(row0, R)], bufs[j],
                                 ld_sems[j])
            else:
                j = b - 2

                @pl.when(t + 1 < NSUB)
                def _next():
                    wait_st(j)
                    pltpu.async_copy(x_hbm.at[j, pl.ds(row0 + R, R)],
                                     bufs[j], ld_sems[j])
            _add_chunk(bufs[b], pos_v)
            if b == BATCH - 1:
                @pl.when(t + 1 < NSUB)
                def _pos_next():
                    pltpu.async_copy(w_hbm.at[pl.ds(row0 + R, R)], pos_v,
                                     sem_p)
            pltpu.async_copy(bufs[b], out_hbm.at[b, pl.ds(row0, R)],
                             st_sems[b])
        return carry

    lax.fori_loop(0, NSUB, t_body, 0)
    for k in range(BATCH):
        wait_st(k)


@jax.jit
def kernel(x, pos_embed_weight):
    mesh = plsc.VectorSubcoreMesh(core_axis_name="c", subcore_axis_name="s",
                                  num_cores=NC, num_subcores=NS)
    run = functools.partial(
        pl.kernel,
        out_type=jax.ShapeDtypeStruct((BATCH, SEQ, DIM), jnp.float32),
        mesh=mesh,
        scratch_types=[
            pltpu.VMEM((R, DIM), jnp.float32),
            pltpu.VMEM((R, DIM), jnp.float32),
            pltpu.VMEM((R, DIM), jnp.float32),
            pltpu.VMEM((R, DIM), jnp.float32),
            pltpu.VMEM((R, DIM), jnp.float32),
            pltpu.SemaphoreType.DMA,
            pltpu.SemaphoreType.DMA,
            pltpu.SemaphoreType.DMA,
            pltpu.SemaphoreType.DMA,
            pltpu.SemaphoreType.DMA,
            pltpu.SemaphoreType.DMA,
            pltpu.SemaphoreType.DMA,
            pltpu.SemaphoreType.DMA,
            pltpu.SemaphoreType.DMA,
        ],
    )(_sc_kernel)
    return run(x, pos_embed_weight)
